# baseline (speedup 1.0000x reference)
import functools

import jax
import jax.numpy as jnp
from jax.experimental import pallas as pl
from jax.experimental.pallas import tpu as pltpu


def _trunk_kernel(
    x_ref,
    w1_ref, b1_ref,
    w2_ref, b2_ref,
    w3_ref, b3_ref, s3_ref,
    g_ref,
    *, n_chunk: int, n_valid: int,
):
    bf = jnp.bfloat16
    n_pad = x_ref.shape[2]

    xbf = x_ref[0].astype(bf)
    z1 = jnp.dot(w1_ref[...], xbf,
                 preferred_element_type=jnp.float32) + b1_ref[...]
    h1 = jnp.maximum(z1, 0.0).astype(bf)
    z2 = jnp.dot(w2_ref[...], h1,
                 preferred_element_type=jnp.float32) + b2_ref[...]
    h2 = jnp.maximum(z2, 0.0).astype(bf)

    zmax = None
    zmin = None
    for c in range(n_pad // n_chunk):
        hc = h2[:, c * n_chunk:(c + 1) * n_chunk]
        z3 = jnp.dot(w3_ref[...], hc,
                     preferred_element_type=jnp.float32) + b3_ref[...]
        if c * n_chunk + n_chunk > n_valid:
            col = c * n_chunk + jax.lax.broadcasted_iota(
                jnp.int32, (1, n_chunk), 1)
            valid = col < n_valid
            cmax = jnp.max(jnp.where(valid, z3, -jnp.inf),
                           axis=-1, keepdims=True)
            cmin = jnp.min(jnp.where(valid, z3, jnp.inf),
                           axis=-1, keepdims=True)
        else:
            cmax = jnp.max(z3, axis=-1, keepdims=True)
            cmin = jnp.min(z3, axis=-1, keepdims=True)
        zmax = cmax if zmax is None else jnp.maximum(zmax, cmax)
        zmin = cmin if zmin is None else jnp.minimum(zmin, cmin)

    g_ref[0] = jnp.maximum(jnp.where(s3_ref[...] >= 0.0, zmax, zmin), 0.0)


def _head_kernel(g_ref, w4_ref, b4_ref, w5_ref, b5_ref, w6_ref, b6_ref,
                 m_ref):
    bf = jnp.bfloat16
    gb = g_ref[...].astype(bf)
    f4 = jnp.maximum(
        jnp.dot(gb, w4_ref[...], preferred_element_type=jnp.float32)
        + b4_ref[...], 0.0).astype(bf)
    f5 = jnp.maximum(
        jnp.dot(f4, w5_ref[...], preferred_element_type=jnp.float32)
        + b5_ref[...], 0.0).astype(bf)
    m_ref[...] = jnp.dot(
        f5, w6_ref[...], preferred_element_type=jnp.float32) + b6_ref[...]


def _apply_kernel(m_ref, x_ref, o_ref):
    bf = jnp.bfloat16
    o_ref[0] = jnp.dot(m_ref[0].astype(bf), x_ref[0].astype(bf),
                       preferred_element_type=jnp.float32)


def kernel(x,
           w1, b1, s1, t1,
           w2, b2, s2, t2,
           w3, b3, s3, t3,
           w4, b4, s4, t4,
           w5, b5, s5, t5,
           w6, b6i):
    B, K, N = x.shape
    c_hidden = w3.shape[0]
    kk = w6.shape[1]
    bf = jnp.bfloat16

    w1b = w1.astype(bf)
    w2f = (w2 * s1.reshape(1, -1)).astype(bf)
    b2f = b2 + w2 @ t1
    w3f = (w3 * s2.reshape(1, -1)).astype(bf)
    b3f = b3 + w3 @ t2
    w4h = (w4 * s3.reshape(-1, 1)).astype(bf)
    b4h = b4 + t3.reshape(1, -1) @ w4
    w5h = (w5 * s4.reshape(-1, 1)).astype(bf)
    b5h = b5 + t4 @ w5
    w6h = (w6 * s5.reshape(-1, 1)).astype(bf)
    b6h = b6i + t5 @ w6

    n_chunk = min(1024, ((N + 127) // 128) * 128)
    n_pad = ((N + n_chunk - 1) // n_chunk) * n_chunk
    x_pad = jnp.pad(x, ((0, 0), (0, 0), (0, n_pad - N))) if n_pad != N else x

    trunk = functools.partial(_trunk_kernel, n_chunk=n_chunk, n_valid=N)
    tparams = (w1b, b1, w2f, b2f, w3f, b3f, s3)
    g = pl.pallas_call(
        trunk,
        out_shape=jax.ShapeDtypeStruct((B, c_hidden, 1), jnp.float32),
        grid=(B,),
        in_specs=[pl.BlockSpec((1, K, n_pad), lambda b: (b, 0, 0))]
        + [pl.BlockSpec(p.shape, lambda b: (0,) * p.ndim) for p in tparams],
        out_specs=pl.BlockSpec((1, c_hidden, 1), lambda b: (b, 0, 0)),
        compiler_params=pltpu.CompilerParams(
            dimension_semantics=("parallel",)),
    )(x_pad, *tparams)

    gmat = g.reshape(B, c_hidden)

    hparams = (w4h, b4h, w5h, b5h, w6h, b6h)
    m = pl.pallas_call(
        _head_kernel,
        out_shape=jax.ShapeDtypeStruct((B, kk), jnp.float32),
        grid=(1,),
        in_specs=[pl.BlockSpec((B, c_hidden), lambda i: (0, 0))]
        + [pl.BlockSpec(p.shape, lambda i: (0, 0)) for p in hparams],
        out_specs=pl.BlockSpec((B, kk), lambda i: (0, 0)),
    )(gmat, *hparams)

    matrix = m.reshape(B, K, K)

    out_pad = pl.pallas_call(
        _apply_kernel,
        out_shape=jax.ShapeDtypeStruct((B, K, n_pad), jnp.float32),
        grid=(B,),
        in_specs=[
            pl.BlockSpec((1, K, K), lambda b: (b, 0, 0)),
            pl.BlockSpec((1, K, n_pad), lambda b: (b, 0, 0)),
        ],
        out_specs=pl.BlockSpec((1, K, n_pad), lambda b: (b, 0, 0)),
        compiler_params=pltpu.CompilerParams(
            dimension_semantics=("parallel",)),
    )(matrix, x_pad)

    return out_pad[:, :, :N] if n_pad != N else out_pad


# speedup vs baseline: 1.0557x; 1.0557x over previous
import functools

import jax
import jax.numpy as jnp
from jax.experimental import pallas as pl
from jax.experimental.pallas import tpu as pltpu


def _fused_stn_kernel(
    x_ref,
    w1_ref, b1_ref,
    w2_ref, b2_ref,
    w3_ref, b3_ref, sgn_ref,
    w4_ref, b4_ref,
    w5_ref, b5_ref,
    w6_ref, b6_ref,
    selc_ref, selr_ref,
    o_ref,
    *, n_chunk: int, n_valid: int, g_batch: int,
):
    bf = jnp.bfloat16
    k_dim = x_ref.shape[1]
    n_pad = x_ref.shape[2]
    c_hidden = w3_ref.shape[0]

    xbfs = []
    g_cols = []
    for g in range(g_batch):
        xbf = x_ref[g].astype(bf)
        xbfs.append(xbf)
        h1 = jnp.maximum(
            jnp.dot(w1_ref[...], xbf, preferred_element_type=jnp.float32)
            + b1_ref[...], 0.0).astype(bf)
        h2 = jnp.maximum(
            jnp.dot(w2_ref[...], h1, preferred_element_type=jnp.float32)
            + b2_ref[...], 0.0).astype(bf)

        emax = None
        for c in range(n_pad // n_chunk):
            hc = h2[:, c * n_chunk:(c + 1) * n_chunk]
            z3 = jnp.dot(w3_ref[...], hc,
                         preferred_element_type=jnp.float32)
            if c * n_chunk + n_chunk > n_valid:
                col = c * n_chunk + jax.lax.broadcasted_iota(
                    jnp.int32, (1, n_chunk), 1)
                z3 = jnp.where(col < n_valid, z3, -jnp.inf)
            cmax = jnp.max(z3, axis=-1, keepdims=True)
            emax = cmax if emax is None else jnp.maximum(emax, cmax)

        r = jnp.maximum(sgn_ref[...] * (emax + b3_ref[...]),
                        0.0).astype(bf)
        g_cols.append(jax.lax.broadcast_in_dim(r, (c_hidden, 128), (0, 1)))

    g_grp = (g_cols[0] if g_batch == 1
             else jnp.concatenate(g_cols, axis=1))

    f4 = jnp.maximum(
        jnp.dot(w4_ref[...], g_grp, preferred_element_type=jnp.float32)
        + b4_ref[...], 0.0).astype(bf)
    f5 = jnp.maximum(
        jnp.dot(w5_ref[...], f4, preferred_element_type=jnp.float32)
        + b5_ref[...], 0.0).astype(bf)
    f5s = (f5[:, :k_dim] if g_batch == 1 else jnp.concatenate(
        [f5[:, 128 * g:128 * g + k_dim] for g in range(g_batch)], axis=1))
    m = (jnp.dot(w6_ref[...], f5s, preferred_element_type=jnp.float32)
         + b6_ref[...]).astype(bf)

    for g in range(g_batch):
        z = m[:, k_dim * g:k_dim * (g + 1)] * selc_ref[...]
        mat = jnp.dot(selr_ref[...], z,
                      preferred_element_type=jnp.float32).astype(bf)
        o_ref[g] = jnp.dot(mat, xbfs[g],
                           preferred_element_type=jnp.float32)


def kernel(x,
           w1, b1, s1, t1,
           w2, b2, s2, t2,
           w3, b3, s3, t3,
           w4, b4, s4, t4,
           w5, b5, s5, t5,
           w6, b6i):
    B, K, N = x.shape
    kk = w6.shape[1]
    bf = jnp.bfloat16

    sgn = jnp.where(s3 >= 0.0, 1.0, -1.0)
    w1b = w1.astype(bf)
    b1b = b1
    w2f = (w2 * s1.reshape(1, -1)).astype(bf)
    b2f = b2 + w2 @ t1
    w3f = (w3 * s2.reshape(1, -1) * sgn).astype(bf)
    b3f = (b3 + w3 @ t2) * sgn
    w4t = w4.T
    w4f = (w4t * s3.reshape(1, -1)).astype(bf)
    b4f = b4.reshape(-1, 1) + w4t @ t3
    w5t = (w5 * s4.reshape(-1, 1)).T.astype(bf)
    b5t = (b5 + t4 @ w5).reshape(-1, 1)
    w6t = (w6 * s5.reshape(-1, 1)).T.astype(bf)
    b6t = (b6i + t5 @ w6).reshape(-1, 1)

    rr = jnp.arange(kk)
    selc = (rr.reshape(-1, 1) % K == jnp.arange(K).reshape(1, -1)).astype(bf)
    selr = (jnp.arange(K).reshape(-1, 1) == rr.reshape(1, -1) // K).astype(bf)

    g_batch = 4
    while B % g_batch:
        g_batch //= 2
    n_chunk = min(1024, ((N + 127) // 128) * 128)
    n_pad = ((N + n_chunk - 1) // n_chunk) * n_chunk
    x_pad = jnp.pad(x, ((0, 0), (0, 0), (0, n_pad - N))) if n_pad != N else x

    body = functools.partial(_fused_stn_kernel, n_chunk=n_chunk,
                             n_valid=N, g_batch=g_batch)
    params = (w1b, b1b, w2f, b2f, w3f, b3f, sgn,
              w4f, b4f, w5t, b5t, w6t, b6t, selc, selr)
    out_pad = pl.pallas_call(
        body,
        out_shape=jax.ShapeDtypeStruct((B, K, n_pad), jnp.float32),
        grid=(B // g_batch,),
        in_specs=[pl.BlockSpec((g_batch, K, n_pad), lambda b: (b, 0, 0))]
        + [pl.BlockSpec(p.shape, lambda b: (0,) * p.ndim) for p in params],
        out_specs=pl.BlockSpec((g_batch, K, n_pad), lambda b: (b, 0, 0)),
        compiler_params=pltpu.CompilerParams(
            dimension_semantics=("parallel",)),
    )(x_pad, *params)

    return out_pad[:, :, :N] if n_pad != N else out_pad


# speedup vs baseline: 1.1276x; 1.0681x over previous
import functools

import jax
import jax.numpy as jnp
from jax.experimental import pallas as pl
from jax.experimental.pallas import tpu as pltpu


def _dot0(a, b):
    return jax.lax.dot_general(
        a, b, (((0,), (0,)), ((), ())), preferred_element_type=jnp.float32)


def _prep_kernel(
    w1_ref, s1_ref, t1_ref,
    w2_ref, b2_ref, s2_ref, t2_ref,
    w3_ref, b3_ref, s3_ref, t3_ref,
    w4_ref, b4_ref, s4_ref, t4_ref,
    w5_ref, b5_ref, s5_ref, t5_ref,
    w6_ref, b6_ref,
    w1o_ref, w2o_ref, b2o_ref, w3o_ref, b3o_ref, sgn_ref,
    w4o_ref, b4o_ref, w5o_ref, b5o_ref, w6o_ref, b6o_ref,
    selc_ref, selr_ref,
    *, k_dim: int,
):
    bf = jnp.bfloat16
    kk = w6_ref.shape[1]

    w1o_ref[...] = w1_ref[...].astype(bf)
    w2o_ref[...] = (w2_ref[...] * jnp.transpose(s1_ref[...])).astype(bf)
    b2o_ref[...] = b2_ref[...] + jnp.dot(
        w2_ref[...], t1_ref[...], preferred_element_type=jnp.float32)

    sgn = jnp.where(s3_ref[...] >= 0.0, 1.0, -1.0)
    sgn_ref[...] = sgn
    w3o_ref[...] = (w3_ref[...] * jnp.transpose(s2_ref[...]) * sgn).astype(bf)
    b3o_ref[...] = (b3_ref[...] + jnp.dot(
        w3_ref[...], t2_ref[...], preferred_element_type=jnp.float32)) * sgn

    w4o_ref[...] = (w4_ref[...] * s3_ref[...]).astype(bf)
    b4o_ref[...] = (jnp.transpose(b4_ref[...])
                    + _dot0(w4_ref[...], t3_ref[...]))
    w5o_ref[...] = (w5_ref[...] * jnp.transpose(s4_ref[...])).astype(bf)
    b5o_ref[...] = (jnp.transpose(b5_ref[...])
                    + jax.lax.dot_general(
                        w5_ref[...], t4_ref[...], (((0,), (1,)), ((), ())),
                        preferred_element_type=jnp.float32))
    w6o_ref[...] = (w6_ref[...] * jnp.transpose(s5_ref[...])).astype(bf)
    b6o_ref[...] = (jnp.transpose(b6_ref[...])
                    + jax.lax.dot_general(
                        w6_ref[...], t5_ref[...], (((0,), (1,)), ((), ())),
                        preferred_element_type=jnp.float32))

    rc = jax.lax.broadcasted_iota(jnp.int32, (kk, k_dim), 0)
    lc = jax.lax.broadcasted_iota(jnp.int32, (kk, k_dim), 1)
    selc_ref[...] = (rc % k_dim == lc).astype(bf)
    ir = jax.lax.broadcasted_iota(jnp.int32, (k_dim, kk), 0)
    rr = jax.lax.broadcasted_iota(jnp.int32, (k_dim, kk), 1)
    selr_ref[...] = (rr // k_dim == ir).astype(bf)


def _fused_stn_kernel(
    x_ref,
    w1_ref, b1_ref,
    w2_ref, b2_ref,
    w3_ref, b3_ref, sgn_ref,
    w4_ref, b4_ref,
    w5_ref, b5_ref,
    w6_ref, b6_ref,
    selc_ref, selr_ref,
    o_ref,
    *, n_chunk: int, n_valid: int, g_batch: int,
):
    bf = jnp.bfloat16
    k_dim = x_ref.shape[1]
    n_pad = x_ref.shape[2]
    c_hidden = w3_ref.shape[0]

    xbfs = []
    g_cols = []
    for g in range(g_batch):
        xbf = x_ref[g].astype(bf)
        xbfs.append(xbf)
        h1 = jnp.maximum(
            jnp.dot(w1_ref[...], xbf, preferred_element_type=jnp.float32)
            + b1_ref[...], 0.0).astype(bf)
        h2 = jnp.maximum(
            jnp.dot(w2_ref[...], h1, preferred_element_type=jnp.float32)
            + b2_ref[...], 0.0).astype(bf)

        emax = None
        for c in range(n_pad // n_chunk):
            hc = h2[:, c * n_chunk:(c + 1) * n_chunk]
            z3 = jnp.dot(w3_ref[...], hc,
                         preferred_element_type=jnp.float32)
            if c * n_chunk + n_chunk > n_valid:
                col = c * n_chunk + jax.lax.broadcasted_iota(
                    jnp.int32, (1, n_chunk), 1)
                z3 = jnp.where(col < n_valid, z3, -jnp.inf)
            cmax = jnp.max(z3, axis=-1, keepdims=True)
            emax = cmax if emax is None else jnp.maximum(emax, cmax)

        r = jnp.maximum(sgn_ref[...] * (emax + b3_ref[...]),
                        0.0).astype(bf)
        g_cols.append(jax.lax.broadcast_in_dim(r, (c_hidden, 128), (0, 1)))

    g_grp = (g_cols[0] if g_batch == 1
             else jnp.concatenate(g_cols, axis=1))

    f4 = jnp.maximum(_dot0(w4_ref[...], g_grp)
                     + b4_ref[...], 0.0).astype(bf)
    f5 = jnp.maximum(_dot0(w5_ref[...], f4)
                     + b5_ref[...], 0.0).astype(bf)
    f5s = (f5[:, :k_dim] if g_batch == 1 else jnp.concatenate(
        [f5[:, 128 * g:128 * g + k_dim] for g in range(g_batch)], axis=1))
    m = (_dot0(w6_ref[...], f5s) + b6_ref[...]).astype(bf)

    for g in range(g_batch):
        z = m[:, k_dim * g:k_dim * (g + 1)] * selc_ref[...]
        mat = jnp.dot(selr_ref[...], z,
                      preferred_element_type=jnp.float32).astype(bf)
        o_ref[g] = jnp.dot(mat, xbfs[g],
                           preferred_element_type=jnp.float32)


def kernel(x,
           w1, b1, s1, t1,
           w2, b2, s2, t2,
           w3, b3, s3, t3,
           w4, b4, s4, t4,
           w5, b5, s5, t5,
           w6, b6i):
    B, K, N = x.shape
    c2 = w2.shape[0]
    c3 = w3.shape[0]
    c4 = w4.shape[1]
    c5 = w5.shape[1]
    kk = w6.shape[1]
    bf = jnp.bfloat16
    f32 = jnp.float32

    prep_in = (w1, s1, t1, w2, b2, s2, t2, w3, b3, s3, t3,
               w4, b4, s4, t4, w5, b5, s5, t5, w6, b6i)
    prep_out_shapes = [
        jax.ShapeDtypeStruct(w1.shape, bf),
        jax.ShapeDtypeStruct(w2.shape, bf),
        jax.ShapeDtypeStruct((c2, 1), f32),
        jax.ShapeDtypeStruct(w3.shape, bf),
        jax.ShapeDtypeStruct((c3, 1), f32),
        jax.ShapeDtypeStruct((c3, 1), f32),
        jax.ShapeDtypeStruct(w4.shape, bf),
        jax.ShapeDtypeStruct((c4, 1), f32),
        jax.ShapeDtypeStruct(w5.shape, bf),
        jax.ShapeDtypeStruct((c5, 1), f32),
        jax.ShapeDtypeStruct(w6.shape, bf),
        jax.ShapeDtypeStruct((kk, 1), f32),
        jax.ShapeDtypeStruct((kk, K), bf),
        jax.ShapeDtypeStruct((K, kk), bf),
    ]
    prep = pl.pallas_call(
        functools.partial(_prep_kernel, k_dim=K),
        out_shape=prep_out_shapes,
        in_specs=[pl.BlockSpec(p.shape, lambda: (0,) * p.ndim)
                  for p in prep_in],
        out_specs=[pl.BlockSpec(s.shape, lambda: (0,) * len(s.shape))
                   for s in prep_out_shapes],
    )(*prep_in)
    (w1b, w2f, b2f, w3f, b3f, sgn,
     w4s, b4c, w5s, b5c, w6s, b6c, selc, selr) = prep

    g_batch = 4
    while B % g_batch:
        g_batch //= 2
    n_chunk = min(1024, ((N + 127) // 128) * 128)
    n_pad = ((N + n_chunk - 1) // n_chunk) * n_chunk
    x_pad = jnp.pad(x, ((0, 0), (0, 0), (0, n_pad - N))) if n_pad != N else x

    body = functools.partial(_fused_stn_kernel, n_chunk=n_chunk,
                             n_valid=N, g_batch=g_batch)
    params = (w1b, b1, w2f, b2f, w3f, b3f, sgn,
              w4s, b4c, w5s, b5c, w6s, b6c, selc, selr)
    out_pad = pl.pallas_call(
        body,
        out_shape=jax.ShapeDtypeStruct((B, K, n_pad), jnp.float32),
        grid=(B // g_batch,),
        in_specs=[pl.BlockSpec((g_batch, K, n_pad), lambda b: (b, 0, 0))]
        + [pl.BlockSpec(p.shape, lambda b: (0,) * p.ndim) for p in params],
        out_specs=pl.BlockSpec((g_batch, K, n_pad), lambda b: (b, 0, 0)),
        compiler_params=pltpu.CompilerParams(
            dimension_semantics=("parallel",)),
    )(x_pad, *params)

    return out_pad[:, :, :N] if n_pad != N else out_pad


# speedup vs baseline: 1.2102x; 1.0733x over previous
import functools

import jax
import jax.numpy as jnp
from jax.experimental import pallas as pl
from jax.experimental.pallas import tpu as pltpu


def _dot0(a, b):
    return jax.lax.dot_general(
        a, b, (((0,), (0,)), ((), ())), preferred_element_type=jnp.float32)


def _prep_kernel(
    w1_ref, s1_ref, t1_ref,
    w2_ref, b2_ref, s2_ref, t2_ref,
    w3_ref, b3_ref, s3_ref, t3_ref,
    w4_ref, b4_ref, s4_ref, t4_ref,
    w5_ref, b5_ref, s5_ref, t5_ref,
    w6_ref, b6_ref,
    w1o_ref, w2o_ref, b2o_ref, w3o_ref, b3o_ref, sgn_ref,
    w4o_ref, b4o_ref, w5o_ref, b5o_ref, w6o_ref, b6o_ref,
    selc_ref, selr_ref,
    *, k_dim: int,
):
    bf = jnp.bfloat16
    kk = w6_ref.shape[1]

    w1o_ref[...] = w1_ref[...].astype(bf)
    w2o_ref[...] = (w2_ref[...] * jnp.transpose(s1_ref[...])).astype(bf)
    b2o_ref[...] = b2_ref[...] + jnp.dot(
        w2_ref[...], t1_ref[...], preferred_element_type=jnp.float32)

    sgn = jnp.where(s3_ref[...] >= 0.0, 1.0, -1.0)
    sgn_ref[...] = sgn
    w3o_ref[...] = (w3_ref[...] * jnp.transpose(s2_ref[...]) * sgn).astype(bf)
    b3o_ref[...] = (b3_ref[...] + jnp.dot(
        w3_ref[...], t2_ref[...], preferred_element_type=jnp.float32)) * sgn

    w4o_ref[...] = (w4_ref[...] * s3_ref[...]).astype(bf)
    b4o_ref[...] = (jnp.transpose(b4_ref[...])
                    + _dot0(w4_ref[...], t3_ref[...]))
    w5o_ref[...] = (w5_ref[...] * jnp.transpose(s4_ref[...])).astype(bf)
    b5o_ref[...] = (jnp.transpose(b5_ref[...])
                    + jax.lax.dot_general(
                        w5_ref[...], t4_ref[...], (((0,), (1,)), ((), ())),
                        preferred_element_type=jnp.float32))
    w6o_ref[...] = (w6_ref[...] * jnp.transpose(s5_ref[...])).astype(bf)
    b6o_ref[...] = (jnp.transpose(b6_ref[...])
                    + jax.lax.dot_general(
                        w6_ref[...], t5_ref[...], (((0,), (1,)), ((), ())),
                        preferred_element_type=jnp.float32))

    rc = jax.lax.broadcasted_iota(jnp.int32, (kk, k_dim), 0)
    lc = jax.lax.broadcasted_iota(jnp.int32, (kk, k_dim), 1)
    selc_ref[...] = (rc % k_dim == lc).astype(bf)
    ir = jax.lax.broadcasted_iota(jnp.int32, (k_dim, kk), 0)
    rr = jax.lax.broadcasted_iota(jnp.int32, (k_dim, kk), 1)
    selr_ref[...] = (rr // k_dim == ir).astype(bf)


def _fused_stn_kernel(
    x_ref,
    w1_ref, b1_ref,
    w2_ref, b2_ref,
    w3_ref, b3_ref, sgn_ref,
    w4_ref, b4_ref,
    w5_ref, b5_ref,
    w6_ref, b6_ref,
    selc_ref, selr_ref,
    o_ref,
    *, n_chunk: int, n_valid: int, g_batch: int,
):
    bf = jnp.bfloat16
    k_dim = x_ref.shape[1]
    n_pad = x_ref.shape[2]
    c_hidden = w3_ref.shape[0]

    xbfs = []
    g_cols = []
    for g in range(g_batch):
        xbf = x_ref[g].astype(bf)
        xbfs.append(xbf)
        h1 = jnp.maximum(
            jnp.dot(w1_ref[...], xbf, preferred_element_type=jnp.float32)
            + b1_ref[...], 0.0).astype(bf)
        h2 = jnp.maximum(
            jnp.dot(w2_ref[...], h1, preferred_element_type=jnp.float32)
            + b2_ref[...], 0.0).astype(bf)

        emax = None
        for c in range(n_pad // n_chunk):
            hc = h2[:, c * n_chunk:(c + 1) * n_chunk]
            z3 = jnp.dot(w3_ref[...], hc,
                         preferred_element_type=jnp.float32)
            if c * n_chunk + n_chunk > n_valid:
                col = c * n_chunk + jax.lax.broadcasted_iota(
                    jnp.int32, (1, n_chunk), 1)
                z3 = jnp.where(col < n_valid, z3, -jnp.inf)
            cmax = jnp.max(z3, axis=-1, keepdims=True)
            emax = cmax if emax is None else jnp.maximum(emax, cmax)

        r = jnp.maximum(sgn_ref[...] * (emax + b3_ref[...]),
                        0.0).astype(bf)
        g_cols.append(jax.lax.broadcast_in_dim(r, (c_hidden, 128), (0, 1)))

    g_grp = (g_cols[0] if g_batch == 1
             else jnp.concatenate(g_cols, axis=1))

    f4 = jnp.maximum(_dot0(w4_ref[...], g_grp)
                     + b4_ref[...], 0.0).astype(bf)
    f5 = jnp.maximum(_dot0(w5_ref[...], f4)
                     + b5_ref[...], 0.0).astype(bf)
    f5s = (f5[:, :k_dim] if g_batch == 1 else jnp.concatenate(
        [f5[:, 128 * g:128 * g + k_dim] for g in range(g_batch)], axis=1))
    m = (_dot0(w6_ref[...], f5s) + b6_ref[...]).astype(bf)

    for g in range(g_batch):
        z = m[:, k_dim * g:k_dim * (g + 1)] * selc_ref[...]
        mat = jnp.dot(selr_ref[...], z,
                      preferred_element_type=jnp.float32).astype(bf)
        o_ref[g] = jnp.dot(mat, xbfs[g],
                           preferred_element_type=jnp.float32)


def kernel(x,
           w1, b1, s1, t1,
           w2, b2, s2, t2,
           w3, b3, s3, t3,
           w4, b4, s4, t4,
           w5, b5, s5, t5,
           w6, b6i):
    B, K, N = x.shape
    c2 = w2.shape[0]
    c3 = w3.shape[0]
    c4 = w4.shape[1]
    c5 = w5.shape[1]
    kk = w6.shape[1]
    bf = jnp.bfloat16
    f32 = jnp.float32

    prep_in = (w1, s1, t1, w2, b2, s2, t2, w3, b3, s3, t3,
               w4, b4, s4, t4, w5, b5, s5, t5, w6, b6i)
    prep_out_shapes = [
        jax.ShapeDtypeStruct(w1.shape, bf),
        jax.ShapeDtypeStruct(w2.shape, bf),
        jax.ShapeDtypeStruct((c2, 1), f32),
        jax.ShapeDtypeStruct(w3.shape, bf),
        jax.ShapeDtypeStruct((c3, 1), f32),
        jax.ShapeDtypeStruct((c3, 1), f32),
        jax.ShapeDtypeStruct(w4.shape, bf),
        jax.ShapeDtypeStruct((c4, 1), f32),
        jax.ShapeDtypeStruct(w5.shape, bf),
        jax.ShapeDtypeStruct((c5, 1), f32),
        jax.ShapeDtypeStruct(w6.shape, bf),
        jax.ShapeDtypeStruct((kk, 1), f32),
        jax.ShapeDtypeStruct((kk, K), bf),
        jax.ShapeDtypeStruct((K, kk), bf),
    ]
    prep = pl.pallas_call(
        functools.partial(_prep_kernel, k_dim=K),
        out_shape=prep_out_shapes,
        in_specs=[pl.BlockSpec(p.shape, lambda: (0,) * p.ndim)
                  for p in prep_in],
        out_specs=[pl.BlockSpec(s.shape, lambda: (0,) * len(s.shape))
                   for s in prep_out_shapes],
    )(*prep_in)
    (w1b, w2f, b2f, w3f, b3f, sgn,
     w4s, b4c, w5s, b5c, w6s, b6c, selc, selr) = prep

    g_batch = 8
    while B % g_batch:
        g_batch //= 2
    n_chunk = min(2048, ((N + 127) // 128) * 128)
    n_pad = ((N + n_chunk - 1) // n_chunk) * n_chunk
    x_pad = jnp.pad(x, ((0, 0), (0, 0), (0, n_pad - N))) if n_pad != N else x

    body = functools.partial(_fused_stn_kernel, n_chunk=n_chunk,
                             n_valid=N, g_batch=g_batch)
    params = (w1b, b1, w2f, b2f, w3f, b3f, sgn,
              w4s, b4c, w5s, b5c, w6s, b6c, selc, selr)
    out_pad = pl.pallas_call(
        body,
        out_shape=jax.ShapeDtypeStruct((B, K, n_pad), jnp.float32),
        grid=(B // g_batch,),
        in_specs=[pl.BlockSpec((g_batch, K, n_pad), lambda b: (b, 0, 0))]
        + [pl.BlockSpec(p.shape, lambda b: (0,) * p.ndim) for p in params],
        out_specs=pl.BlockSpec((g_batch, K, n_pad), lambda b: (b, 0, 0)),
        compiler_params=pltpu.CompilerParams(
            dimension_semantics=("parallel",)),
    )(x_pad, *params)

    return out_pad[:, :, :N] if n_pad != N else out_pad
